# baseline (speedup 1.0000x reference)
import functools

import jax
import jax.numpy as jnp
from jax import lax
from jax.experimental import pallas as pl
from jax.experimental.pallas import tpu as pltpu
from jax.experimental.pallas import tpu_sc as plsc

N_NODES = 10000
D = 128
NP = 10240
CHUNK = 48
CPP = 20
NW = 32
ROWS_PER_SUB = NP // 16
RD = 40


def _tc_prep(feat, W, al, ar):
    def body(feat_ref, w_ref, al_ref, ar_ref, fs_ref, elr_ref):
        fs = jnp.dot(feat_ref[...], w_ref[...],
                     preferred_element_type=jnp.float32)
        fs_ref[...] = fs
        el = jnp.sum(fs * al_ref[...], axis=1)
        er = jnp.sum(fs * ar_ref[...], axis=1)
        elr_ref[...] = jnp.stack([el, er], axis=0)

    return pl.pallas_call(
        body,
        out_shape=(
            jax.ShapeDtypeStruct((N_NODES, D), jnp.float32),
            jax.ShapeDtypeStruct((2, N_NODES), jnp.float32),
        ),
    )(feat, W, al, ar)


def _make_sc_edges(num_phases):
    mesh = plsc.VectorSubcoreMesh(core_axis_name="c", subcore_axis_name="s")

    @functools.partial(
        pl.kernel,
        out_type=(
            jax.ShapeDtypeStruct((2, NP, D), jnp.float32),
            jax.ShapeDtypeStruct((NW, NP), jnp.float32),
        ),
        mesh=mesh,
        compiler_params=pltpu.CompilerParams(needs_layout_passes=False),
        scratch_types=[
            pltpu.VMEM((NP,), jnp.float32),
            pltpu.VMEM((NP,), jnp.float32),
            pltpu.VMEM((NP,), jnp.float32),
            pltpu.VMEM((CPP, 2, CHUNK), jnp.int32),
            pltpu.VMEM((2, CHUNK), jnp.int32),
            pltpu.VMEM((2, CHUNK), jnp.int32),
            pltpu.VMEM((CHUNK, D), jnp.float32),
            pltpu.VMEM((CHUNK, D), jnp.float32),
            pltpu.VMEM_SHARED((NP, D), jnp.float32),
            pltpu.SemaphoreType.DMA,
            pltpu.SemaphoreType.DMA,
            pltpu.SemaphoreType.DMA,
            pltpu.SemaphoreType.DMA,
        ],
    )
    def sc_edges(fs_hbm, idx_hbm, el_hbm, er_hbm,
                 rst_out, den_out,
                 el_v, er_v, den_v, stg, sd0, sd1, rows0, rows1, rst_sh,
                 gsem0, gsem1, ssem0, ssem1):
        c = lax.axis_index("c")
        s = lax.axis_index("s")
        wid = s * 2 + c
        sd = (sd0, sd1)
        rows = (rows0, rows1)
        gsem = (gsem0, gsem1)
        ssem = (ssem0, ssem1)

        pltpu.sync_copy(el_hbm, el_v)
        pltpu.sync_copy(er_hbm, er_v)

        zero16 = jnp.zeros((16,), jnp.float32)

        def zden(i, _):
            den_v[pl.ds(i * 16, 16)] = zero16
            return 0
        lax.fori_loop(0, NP // 16, zden, 0)

        def zrow(j, _):
            for k in range(8):
                rows0[j, pl.ds(k * 16, 16)] = zero16
            return 0
        lax.fori_loop(0, CHUNK, zrow, 0)
        for b in range(ROWS_PER_SUB // RD):
            pltpu.sync_copy(
                rows0.at[pl.ds(0, RD), :],
                rst_sh.at[pl.ds(s * ROWS_PER_SUB + b * RD, RD), :])
        plsc.subcore_barrier()

        def load_idx(ci, slot):
            sdb = sd[slot]

            def mv(j, _):
                sl = pl.ds(j * 16, 16)
                sdb[0, sl] = stg[ci, 0, sl]
                sdb[1, sl] = stg[ci, 1, sl]
                return 0
            lax.fori_loop(0, CHUNK // 16, mv, 0)

        def start_gather(slot):
            pltpu.async_copy(fs_hbm.at[sd[slot].at[0]], rows[slot], gsem[slot])

        def wait_gather(slot):
            pltpu.make_async_copy(fs_hbm.at[sd[slot].at[0]], rows[slot],
                                  gsem[slot]).wait()

        def start_scatter(slot):
            pltpu.async_copy(rows[slot], rst_sh.at[sd[slot].at[1]],
                             ssem[slot], add=True)

        def wait_scatter(slot):
            pltpu.make_async_copy(rows[slot], rst_sh.at[sd[slot].at[1]],
                                  ssem[slot]).wait()

        def compute_chunk(slot):
            sdb = sd[slot]
            r = rows[slot]

            def grp(j, _):
                sl = pl.ds(j * 16, 16)
                sidx = sdb[0, sl]
                didx = sdb[1, sl]
                e = (plsc.load_gather(el_v, [sidx])
                     + plsc.load_gather(er_v, [didx]))
                e = jnp.where(e > 0, e, 0.2 * e)
                w16 = jnp.exp(e)
                plsc.addupdate_scatter(den_v, [didx], w16)
                for l in range(16):
                    lane = jnp.full((16,), l, jnp.int32)
                    wj = w16.at[lane].get(mode="promise_in_bounds")
                    row = j * 16 + l
                    for k in range(8):
                        rsl = pl.ds(k * 16, 16)
                        r[row, rsl] = r[row, rsl] * wj
                return 0
            lax.fori_loop(0, CHUNK // 16, grp, 0)

        def step(ci, slot):
            other = 1 - slot
            wait_gather(slot)
            wait_scatter(other)
            load_idx(ci + 1, other)
            start_gather(other)
            compute_chunk(slot)
            start_scatter(slot)

        def pair_body(i, _):
            step(2 * i + 1, 1)
            step(2 * i + 2, 0)
            return 0

        for p in range(num_phases):
            pltpu.sync_copy(idx_hbm.at[wid, p], stg)
            load_idx(0, 0)
            start_gather(0)
            load_idx(1, 1)
            start_gather(1)
            wait_gather(0)
            compute_chunk(0)
            start_scatter(0)
            lax.fori_loop(0, (CPP - 2) // 2, pair_body, 0)
            wait_gather(1)
            wait_scatter(0)
            compute_chunk(1)
            start_scatter(1)
            wait_scatter(1)

        pltpu.sync_copy(den_v, den_out.at[wid])
        plsc.subcore_barrier()

        for b in range(ROWS_PER_SUB // RD):
            r0 = s * ROWS_PER_SUB + b * RD
            pltpu.sync_copy(rst_sh.at[pl.ds(r0, RD), :],
                            rows0.at[pl.ds(0, RD), :])
            pltpu.sync_copy(rows0.at[pl.ds(0, RD), :],
                            rst_out.at[c, pl.ds(r0, RD), :])

    return sc_edges


def _tc_combine(parts, dparts):
    def body(p_ref, d_ref, o_ref):
        den = jnp.sum(d_ref[...], axis=0)
        num = p_ref[0] + p_ref[1]
        o_ref[...] = num[:N_NODES] / (den[:N_NODES, None] + 1e-9)

    return pl.pallas_call(
        body,
        out_shape=jax.ShapeDtypeStruct((N_NODES, D), jnp.float32),
    )(parts, dparts)


def kernel(feat, edge_index, W, attn_l, attn_r):
    num_edges = edge_index.shape[1]
    num_phases = -(-num_edges // (NW * CHUNK * CPP))
    cpt = num_phases * CPP
    ept = cpt * CHUNK
    pad = NW * ept - num_edges

    al = attn_l.reshape(1, D).astype(jnp.float32)
    ar = attn_r.reshape(1, D).astype(jnp.float32)
    fs, elr = _tc_prep(feat.astype(jnp.float32), W.astype(jnp.float32), al, ar)

    zpad = jnp.zeros((NP - N_NODES,), jnp.float32)
    el = jnp.concatenate([elr[0], zpad])
    er = jnp.concatenate([elr[1], zpad])

    src = jnp.concatenate(
        [edge_index[0].astype(jnp.int32), jnp.zeros((pad,), jnp.int32)])
    dst = jnp.concatenate(
        [edge_index[1].astype(jnp.int32),
         jnp.full((pad,), N_NODES, jnp.int32)])
    idx2 = jnp.stack([src.reshape(NW, num_phases, CPP, CHUNK),
                      dst.reshape(NW, num_phases, CPP, CHUNK)], axis=3)

    parts, dparts = _make_sc_edges(num_phases)(fs, idx2, el, er)
    rst = _tc_combine(parts, dparts)
    return rst.reshape(N_NODES, 1, D)


# speedup vs baseline: 1.1273x; 1.1273x over previous
import functools

import jax
import jax.numpy as jnp
from jax import lax
from jax.experimental import pallas as pl
from jax.experimental.pallas import tpu as pltpu
from jax.experimental.pallas import tpu_sc as plsc

N_NODES = 10000
D = 128
NP = 10240
CHUNK = 48
CPP = 36
NW = 32
ROWS_PER_SUB = NP // 16
RD = 40


def _tc_prep(feat, W, al, ar):
    def body(feat_ref, w_ref, al_ref, ar_ref, fs_ref, elr_ref):
        fs = jnp.dot(feat_ref[...], w_ref[...],
                     preferred_element_type=jnp.float32)
        fs_ref[...] = fs
        el = jnp.sum(fs * al_ref[...], axis=1)
        er = jnp.sum(fs * ar_ref[...], axis=1)
        elr_ref[...] = jnp.stack([el, er], axis=0)

    return pl.pallas_call(
        body,
        out_shape=(
            jax.ShapeDtypeStruct((N_NODES, D), jnp.float32),
            jax.ShapeDtypeStruct((2, N_NODES), jnp.float32),
        ),
    )(feat, W, al, ar)


def _make_sc_edges(num_phases):
    mesh = plsc.VectorSubcoreMesh(core_axis_name="c", subcore_axis_name="s")

    @functools.partial(
        pl.kernel,
        out_type=(
            jax.ShapeDtypeStruct((2, NP, D), jnp.float32),
            jax.ShapeDtypeStruct((NW, NP), jnp.float32),
        ),
        mesh=mesh,
        compiler_params=pltpu.CompilerParams(needs_layout_passes=False),
        scratch_types=[
            pltpu.VMEM((NP,), jnp.float32),
            pltpu.VMEM((NP,), jnp.float32),
            pltpu.VMEM((NP,), jnp.float32),
            pltpu.VMEM((CPP * 2 * CHUNK,), jnp.int32),
            pltpu.VMEM((2, CHUNK), jnp.int32),
            pltpu.VMEM((2, CHUNK), jnp.int32),
            pltpu.VMEM((CHUNK, D), jnp.float32),
            pltpu.VMEM((CHUNK, D), jnp.float32),
            pltpu.VMEM_SHARED((NP, D), jnp.float32),
            pltpu.SemaphoreType.DMA,
            pltpu.SemaphoreType.DMA,
            pltpu.SemaphoreType.DMA,
            pltpu.SemaphoreType.DMA,
        ],
    )
    def sc_edges(fs_hbm, idx_hbm, el_hbm, er_hbm,
                 rst_out, den_out,
                 el_v, er_v, den_v, stg, sd0, sd1, rows0, rows1, rst_sh,
                 gsem0, gsem1, ssem0, ssem1):
        c = lax.axis_index("c")
        s = lax.axis_index("s")
        wid = s * 2 + c
        sd = (sd0, sd1)
        rows = (rows0, rows1)
        gsem = (gsem0, gsem1)
        ssem = (ssem0, ssem1)

        pltpu.sync_copy(el_hbm, el_v)
        pltpu.sync_copy(er_hbm, er_v)

        zero16 = jnp.zeros((16,), jnp.float32)

        def zden(i, _):
            den_v[pl.ds(i * 16, 16)] = zero16
            return 0
        lax.fori_loop(0, NP // 16, zden, 0)

        def zrow(j, _):
            for k in range(8):
                rows0[j, pl.ds(k * 16, 16)] = zero16
            return 0
        lax.fori_loop(0, CHUNK, zrow, 0)
        for b in range(ROWS_PER_SUB // RD):
            pltpu.sync_copy(
                rows0.at[pl.ds(0, RD), :],
                rst_sh.at[pl.ds(s * ROWS_PER_SUB + b * RD, RD), :])
        plsc.subcore_barrier()

        def load_idx(ci, slot):
            sdb = sd[slot]
            base = ci * 2 * CHUNK

            def mv(j, _):
                sl = pl.ds(j * 16, 16)
                sdb[0, sl] = stg[pl.ds(base + j * 16, 16)]
                sdb[1, sl] = stg[pl.ds(base + CHUNK + j * 16, 16)]
                return 0
            lax.fori_loop(0, CHUNK // 16, mv, 0)

        def start_gather(slot):
            pltpu.async_copy(fs_hbm.at[sd[slot].at[0]], rows[slot], gsem[slot])

        def wait_gather(slot):
            pltpu.make_async_copy(fs_hbm.at[sd[slot].at[0]], rows[slot],
                                  gsem[slot]).wait()

        def start_scatter(slot):
            pltpu.async_copy(rows[slot], rst_sh.at[sd[slot].at[1]],
                             ssem[slot], add=True)

        def wait_scatter(slot):
            pltpu.make_async_copy(rows[slot], rst_sh.at[sd[slot].at[1]],
                                  ssem[slot]).wait()

        def compute_chunk(slot):
            sdb = sd[slot]
            r = rows[slot]

            def grp(j, _):
                sl = pl.ds(j * 16, 16)
                sidx = sdb[0, sl]
                didx = sdb[1, sl]
                e = (plsc.load_gather(el_v, [sidx])
                     + plsc.load_gather(er_v, [didx]))
                e = jnp.where(e > 0, e, 0.2 * e)
                w16 = jnp.exp(e)
                plsc.addupdate_scatter(den_v, [didx], w16)
                for l in range(16):
                    lane = jnp.full((16,), l, jnp.int32)
                    wj = w16.at[lane].get(mode="promise_in_bounds")
                    row = j * 16 + l
                    for k in range(8):
                        rsl = pl.ds(k * 16, 16)
                        r[row, rsl] = r[row, rsl] * wj
                return 0
            lax.fori_loop(0, CHUNK // 16, grp, 0)

        def step(ci, slot):
            other = 1 - slot
            wait_gather(slot)
            wait_scatter(other)
            load_idx(ci + 1, other)
            start_gather(other)
            compute_chunk(slot)
            start_scatter(slot)

        def pair_body(i, _):
            step(2 * i + 1, 1)
            step(2 * i + 2, 0)
            return 0

        for p in range(num_phases):
            pltpu.sync_copy(idx_hbm.at[wid, p], stg)
            load_idx(0, 0)
            start_gather(0)
            load_idx(1, 1)
            start_gather(1)
            wait_gather(0)
            compute_chunk(0)
            start_scatter(0)
            lax.fori_loop(0, (CPP - 2) // 2, pair_body, 0)
            wait_gather(1)
            wait_scatter(0)
            compute_chunk(1)
            start_scatter(1)
            wait_scatter(1)

        pltpu.sync_copy(den_v, den_out.at[wid])
        plsc.subcore_barrier()

        for b in range(ROWS_PER_SUB // RD):
            r0 = s * ROWS_PER_SUB + b * RD
            pltpu.sync_copy(rst_sh.at[pl.ds(r0, RD), :],
                            rows0.at[pl.ds(0, RD), :])
            pltpu.sync_copy(rows0.at[pl.ds(0, RD), :],
                            rst_out.at[c, pl.ds(r0, RD), :])

    return sc_edges


def _tc_combine(parts, dparts):
    def body(p_ref, d_ref, o_ref):
        den = jnp.sum(d_ref[...], axis=0)
        num = p_ref[0] + p_ref[1]
        o_ref[...] = num[:N_NODES] / (den[:N_NODES, None] + 1e-9)

    return pl.pallas_call(
        body,
        out_shape=jax.ShapeDtypeStruct((N_NODES, D), jnp.float32),
    )(parts, dparts)


def kernel(feat, edge_index, W, attn_l, attn_r):
    num_edges = edge_index.shape[1]
    num_phases = -(-num_edges // (NW * CHUNK * CPP))
    cpt = num_phases * CPP
    ept = cpt * CHUNK
    pad = NW * ept - num_edges

    al = attn_l.reshape(1, D).astype(jnp.float32)
    ar = attn_r.reshape(1, D).astype(jnp.float32)
    fs, elr = _tc_prep(feat.astype(jnp.float32), W.astype(jnp.float32), al, ar)

    zpad = jnp.zeros((NP - N_NODES,), jnp.float32)
    el = jnp.concatenate([elr[0], zpad])
    er = jnp.concatenate([elr[1], zpad])

    src = jnp.concatenate(
        [edge_index[0].astype(jnp.int32), jnp.zeros((pad,), jnp.int32)])
    dst = jnp.concatenate(
        [edge_index[1].astype(jnp.int32),
         jnp.full((pad,), N_NODES, jnp.int32)])
    idx2 = jnp.stack([src.reshape(NW, num_phases, CPP, CHUNK),
                      dst.reshape(NW, num_phases, CPP, CHUNK)],
                     axis=3).reshape(NW, num_phases, CPP * 2 * CHUNK)

    parts, dparts = _make_sc_edges(num_phases)(fs, idx2, el, er)
    rst = _tc_combine(parts, dparts)
    return rst.reshape(N_NODES, 1, D)
